# baseline (speedup 1.0000x reference)
import jax
import jax.numpy as jnp
from jax.experimental import pallas as pl
from jax.experimental.pallas import tpu as pltpu

_BM = 1024


def _moe_body(x_ref, w_hbm, b_ref, gw_ref, gb_ref, o_ref, w_vmem, sems):
    step = pl.program_id(0)
    E = gw_ref.shape[0]

    @pl.when(step == 0)
    def _start_weight_copies():
        for e in range(E):
            pltpu.make_async_copy(w_hbm.at[e], w_vmem.at[e], sems.at[e]).start()

    xb = x_ref[...]
    scores = jax.lax.dot_general(
        xb, gw_ref[...], (((1,), (1,)), ((), ()))) + gb_ref[...]
    eidx = jax.lax.broadcasted_iota(jnp.int32, scores.shape, 1)
    m1 = jnp.max(scores, axis=1)
    i1 = jnp.min(jnp.where(scores == m1[:, None], eidx, E), axis=1)
    sel1 = eidx == i1[:, None]
    masked = jnp.where(sel1, -jnp.inf, scores)
    m2 = jnp.max(masked, axis=1)
    i2 = jnp.min(jnp.where(masked == m2[:, None], eidx, E), axis=1)
    sel2 = eidx == i2[:, None]
    e2 = jnp.exp(m2 - m1)
    denom = 1.0 + e2
    w1 = (1.0 / denom)[:, None]
    w2 = (e2 / denom)[:, None]
    g = jnp.where(sel1, w1, 0.0) + jnp.where(sel2, w2, 0.0)
    acc = jax.lax.dot_general(g, b_ref[...], (((1,), (0,)), ((), ())))
    for e in range(E):
        @pl.when(step == 0)
        def _wait_weights(e=e):
            pltpu.make_async_copy(w_hbm.at[e], w_vmem.at[e], sems.at[e]).wait()

        ye = jax.lax.dot_general(
            xb, w_vmem[e], (((1,), (1,)), ((), ())))
        acc = acc + g[:, e:e + 1] * ye
    o_ref[...] = acc


def kernel(x, expert_w, expert_b, gate_w, gate_b):
    B, S, D = x.shape
    E, O, _ = expert_w.shape
    total = B * S
    x2 = x.reshape(total, D)
    gb2 = gate_b.reshape(1, E)
    out = pl.pallas_call(
        _moe_body,
        grid=(total // _BM,),
        in_specs=[
            pl.BlockSpec((_BM, D), lambda i: (i, 0)),
            pl.BlockSpec(memory_space=pl.ANY),
            pl.BlockSpec((E, O), lambda i: (0, 0)),
            pl.BlockSpec((E, D), lambda i: (0, 0)),
            pl.BlockSpec((1, E), lambda i: (0, 0)),
        ],
        out_specs=pl.BlockSpec((_BM, O), lambda i: (i, 0)),
        out_shape=jax.ShapeDtypeStruct((total, O), jnp.float32),
        scratch_shapes=[
            pltpu.VMEM((E, O, D), jnp.float32),
            pltpu.SemaphoreType.DMA((E,)),
        ],
    )(x2, expert_w, expert_b, gate_w, gb2)
    return out.reshape(B, S, O)


# speedup vs baseline: 1.3993x; 1.3993x over previous
import jax
import jax.numpy as jnp
from jax.experimental import pallas as pl

_BM = 1024


def _moe_body(x_ref, w_ref, b_ref, gw_ref, gb_ref, o_ref):
    xb = x_ref[...]
    E = gw_ref.shape[0]
    scores = jax.lax.dot_general(
        xb, gw_ref[...], (((1,), (1,)), ((), ()))) + gb_ref[...]
    eidx = jax.lax.broadcasted_iota(jnp.int32, scores.shape, 1)
    m1 = jnp.max(scores, axis=1)
    i1 = jnp.min(jnp.where(scores == m1[:, None], eidx, E), axis=1)
    sel1 = eidx == i1[:, None]
    masked = jnp.where(sel1, -jnp.inf, scores)
    m2 = jnp.max(masked, axis=1)
    i2 = jnp.min(jnp.where(masked == m2[:, None], eidx, E), axis=1)
    sel2 = eidx == i2[:, None]
    e2 = jnp.exp(m2 - m1)
    denom = 1.0 + e2
    w1 = (1.0 / denom)[:, None]
    w2 = (e2 / denom)[:, None]
    g = jnp.where(sel1, w1, 0.0) + jnp.where(sel2, w2, 0.0)
    acc = jax.lax.dot_general(g, b_ref[...], (((1,), (0,)), ((), ())))
    for e in range(E):
        ye = jax.lax.dot_general(
            xb, w_ref[e], (((1,), (1,)), ((), ())))
        acc = acc + g[:, e:e + 1] * ye
    o_ref[...] = acc


def kernel(x, expert_w, expert_b, gate_w, gate_b):
    B, S, D = x.shape
    E, O, _ = expert_w.shape
    total = B * S
    x2 = x.reshape(total, D)
    gb2 = gate_b.reshape(1, E)
    out = pl.pallas_call(
        _moe_body,
        grid=(total // _BM,),
        in_specs=[
            pl.BlockSpec((_BM, D), lambda i: (i, 0)),
            pl.BlockSpec((E, O, D), lambda i: (0, 0, 0)),
            pl.BlockSpec((E, O), lambda i: (0, 0)),
            pl.BlockSpec((E, D), lambda i: (0, 0)),
            pl.BlockSpec((1, E), lambda i: (0, 0)),
        ],
        out_specs=pl.BlockSpec((_BM, O), lambda i: (i, 0)),
        out_shape=jax.ShapeDtypeStruct((total, O), jnp.float32),
    )(x2, expert_w, expert_b, gate_w, gb2)
    return out.reshape(B, S, O)
